# baseline (speedup 1.0000x reference)
import functools

import jax
import jax.numpy as jnp
from jax import lax
from jax.experimental import pallas as pl
from jax.experimental.pallas import tpu as pltpu
from jax.experimental.pallas import tpu_sc as plsc

_B, _L, _E = 4096, 200, 128
_H, _C = 256, 128
_NC, _NS = 2, 16
_NW = _NC * _NS
_RPW = _B // _NW
_HALF = _L // 2
_QTR = _L // 4
_RING = 8


def _sc_pool(idx_w, idx_b, idx_t, emb_w, emb_b, emb_t):
    mesh = plsc.VectorSubcoreMesh(core_axis_name="c", subcore_axis_name="s")
    out_t = [jax.ShapeDtypeStruct((_B, _E), jnp.float32) for _ in range(3)]
    _UPR = _L // _QTR
    _NU = _UPR * _RPW
    scratch = (
        [pltpu.VMEM((_NU, _QTR), jnp.int32)]
        + [pltpu.VMEM((_QTR, _E), jnp.float32) for _ in range(_RING)]
        + [pltpu.VMEM((16, _E), jnp.float32)]
        + [pltpu.SemaphoreType.DMA for _ in range(_RING)]
    )

    @functools.partial(pl.kernel, mesh=mesh, out_type=out_t, scratch_types=scratch)
    def k(iw, ib, it, ew, eb, et, ow, ob, ot, idx_v, *rest):
        bufs = rest[:_RING]
        sums = rest[_RING]
        sems = rest[_RING + 1:]
        wid = lax.axis_index("s") * _NC + lax.axis_index("c")
        base = wid * _RPW

        for idx_hbm, tab, out_hbm in ((iw, ew, ow), (ib, eb, ob), (it, et, ot)):
            pltpu.sync_copy(idx_hbm.at[pl.ds(_UPR * base, _NU)], idx_v)

            def _start_g0(r, s, tab=tab):
                pltpu.make_async_copy(
                    tab.at[idx_v.at[_UPR * r]], bufs[s], sems[s]).start()

            def _start_adds(r, s, tab=tab):
                for c in range(1, _UPR):
                    pltpu.make_async_copy(
                        tab.at[idx_v.at[_UPR * r + c]], bufs[s], sems[s]
                    ).start(add=True)

            def _wait(s, n, tab=tab):
                for _ in range(n):
                    pltpu.make_async_copy(
                        tab.at[idx_v.at[0]], bufs[s], sems[s]).wait()

            def _reduce(s, out_slot):
                def lbody(l, a):
                    return tuple(a[v] + bufs[s][l, pl.ds(16 * v, 16)]
                                 for v in range(8))
                accs = lax.fori_loop(
                    0, _QTR, lbody,
                    tuple(jnp.zeros((16,), jnp.float32) for _ in range(8)),
                    unroll=5)
                for v in range(8):
                    sums[out_slot, pl.ds(16 * v, 16)] = accs[v]

            for r in range(4):
                _start_g0(r, r)
            for r in range(2):
                _wait(r, 1)
                _start_adds(r, r)

            def body(i, carry):
                for j in range(_RING):
                    t = _RING * i + j

                    @pl.when(t + 4 < _RPW)
                    def _(t=t, j=j):
                        _start_g0(t + 4, (j + 4) % _RING)

                    @pl.when(t + 2 < _RPW)
                    def _(t=t, j=j):
                        s = (j + 2) % _RING
                        _wait(s, 1)
                        _start_adds(t + 2, s)

                    _wait(j, _UPR - 1)
                    _reduce(j, lax.rem(t, 16))

                @pl.when(lax.rem(i, 2) == 1)
                def _(i=i, out_hbm=out_hbm):
                    off = pl.multiple_of(base + _RING * i - 8, 16)
                    pltpu.sync_copy(sums, out_hbm.at[pl.ds(off, 16)])
                return carry

            lax.fori_loop(0, _RPW // _RING, body, 0)

    return k(idx_w, idx_b, idx_t, emb_w, emb_b, emb_t)


def _mlp(sw, sb, st, W1, b1, W2, b2):
    w1w = W1[:, 0:_E].T
    w1b = W1[:, _E:2 * _E].T
    w1t = W1[:, 2 * _E:3 * _E].T
    w2t = W2.T
    b1r = b1.reshape(1, _H)
    b2r = b2.reshape(1, _C)
    blk = 1024

    def body(swr, sbr, strr, w1wr, w1br, w1tr, b1r_, w2r, b2r_, outr):
        scale = jnp.float32(1.0 / _L)
        h = jnp.dot(swr[...] * scale, w1wr[...], preferred_element_type=jnp.float32)
        h = h + jnp.dot(sbr[...] * scale, w1br[...], preferred_element_type=jnp.float32)
        h = h + jnp.dot(strr[...] * scale, w1tr[...], preferred_element_type=jnp.float32)
        h = jnp.maximum(h + b1r_[...], 0.0)
        outr[...] = jnp.dot(h, w2r[...], preferred_element_type=jnp.float32) + b2r_[...]

    return pl.pallas_call(
        body,
        grid=(_B // blk,),
        in_specs=[
            pl.BlockSpec((blk, _E), lambda i: (i, 0)),
            pl.BlockSpec((blk, _E), lambda i: (i, 0)),
            pl.BlockSpec((blk, _E), lambda i: (i, 0)),
            pl.BlockSpec((_E, _H), lambda i: (0, 0)),
            pl.BlockSpec((_E, _H), lambda i: (0, 0)),
            pl.BlockSpec((_E, _H), lambda i: (0, 0)),
            pl.BlockSpec((1, _H), lambda i: (0, 0)),
            pl.BlockSpec((_H, _C), lambda i: (0, 0)),
            pl.BlockSpec((1, _C), lambda i: (0, 0)),
        ],
        out_specs=pl.BlockSpec((blk, _C), lambda i: (i, 0)),
        out_shape=jax.ShapeDtypeStruct((_B, _C), jnp.float32),
    )(sw, sb, st, w1w, w1b, w1t, b1r, w2t, b2r)


def kernel(inputs, bigram, trigram, emb_word, emb_bi, emb_tri, W1, b1, W2, b2):
    iw = inputs.astype(jnp.int32).reshape(4 * _B, _QTR)
    ib = bigram.astype(jnp.int32).reshape(4 * _B, _QTR)
    it = trigram.astype(jnp.int32).reshape(4 * _B, _QTR)
    sw, sb, st = _sc_pool(iw, ib, it, emb_word, emb_bi, emb_tri)
    return _mlp(sw, sb, st, W1, b1, W2, b2)


# speedup vs baseline: 1.0785x; 1.0785x over previous
import functools

import jax
import jax.numpy as jnp
from jax import lax
from jax.experimental import pallas as pl
from jax.experimental.pallas import tpu as pltpu
from jax.experimental.pallas import tpu_sc as plsc

_B, _L, _E = 4096, 200, 128
_H, _C = 256, 128
_NC, _NS = 2, 16
_NW = _NC * _NS
_RPW = _B // _NW
_HALF = _L // 2
_QTR = _L // 4
_RING = 8


def _sc_pool(idx_w, idx_b, idx_t, emb_w, emb_b, emb_t):
    mesh = plsc.VectorSubcoreMesh(core_axis_name="c", subcore_axis_name="s")
    out_t = [jax.ShapeDtypeStruct((_B, _E), jnp.float32) for _ in range(3)]
    _UPR = _L // _QTR
    _NU = _UPR * _RPW
    scratch = (
        [pltpu.VMEM((_NU, _QTR), jnp.int32)]
        + [pltpu.VMEM((_QTR, _E), jnp.float32) for _ in range(_RING)]
        + [pltpu.VMEM((16, _E), jnp.float32)]
        + [pltpu.SemaphoreType.DMA for _ in range(_RING)]
    )

    @functools.partial(pl.kernel, mesh=mesh, out_type=out_t, scratch_types=scratch)
    def k(iw, ib, it, ew, eb, et, ow, ob, ot, idx_v, *rest):
        bufs = rest[:_RING]
        sums = rest[_RING]
        sems = rest[_RING + 1:]
        wid = lax.axis_index("s") * _NC + lax.axis_index("c")
        base = wid * _RPW

        for idx_hbm, tab, out_hbm in ((iw, ew, ow), (ib, eb, ob), (it, et, ot)):
            pltpu.sync_copy(idx_hbm.at[pl.ds(_UPR * base, _NU)], idx_v)

            def _start(u, b, tab=tab):
                pltpu.make_async_copy(tab.at[idx_v.at[u]], bufs[b], sems[b]).start()

            def _wait(b, tab=tab):
                pltpu.make_async_copy(tab.at[idx_v.at[0]], bufs[b], sems[b]).wait()

            def _unit_acc(b, accs):
                def lbody(l, a):
                    return tuple(a[v] + bufs[b][l, pl.ds(16 * v, 16)]
                                 for v in range(8))
                return lax.fori_loop(0, _QTR, lbody, accs, unroll=10)

            def _store(slot, accs):
                for v in range(8):
                    sums[slot, pl.ds(16 * v, 16)] = accs[v]

            _zeros = tuple(jnp.zeros((16,), jnp.float32) for _ in range(8))
            for b in range(_RING):
                _start(b, b)

            def body(i, carry):
                u = _RING * i
                accs = None
                for b in range(_RING):
                    _wait(b)
                    accs = _unit_acc(b, _zeros if b % _UPR == 0 else accs)
                    if b % _UPR == _UPR - 1:
                        _store(lax.rem(2 * i + b // _UPR, 16), accs)

                    @pl.when(u + _RING + b < _NU)
                    def _(u=u, b=b):
                        _start(u + _RING + b, b)

                @pl.when(lax.rem(i, 8) == 7)
                def _(out_hbm=out_hbm):
                    off = pl.multiple_of(base + 2 * i - 14, 16)
                    pltpu.sync_copy(sums, out_hbm.at[pl.ds(off, 16)])
                return carry

            lax.fori_loop(0, _NU // _RING, body, 0)

    return k(idx_w, idx_b, idx_t, emb_w, emb_b, emb_t)


def _mlp(sw, sb, st, W1, b1, W2, b2):
    w1w = W1[:, 0:_E].T
    w1b = W1[:, _E:2 * _E].T
    w1t = W1[:, 2 * _E:3 * _E].T
    w2t = W2.T
    b1r = b1.reshape(1, _H)
    b2r = b2.reshape(1, _C)
    blk = 1024

    def body(swr, sbr, strr, w1wr, w1br, w1tr, b1r_, w2r, b2r_, outr):
        scale = jnp.float32(1.0 / _L)
        h = jnp.dot(swr[...] * scale, w1wr[...], preferred_element_type=jnp.float32)
        h = h + jnp.dot(sbr[...] * scale, w1br[...], preferred_element_type=jnp.float32)
        h = h + jnp.dot(strr[...] * scale, w1tr[...], preferred_element_type=jnp.float32)
        h = jnp.maximum(h + b1r_[...], 0.0)
        outr[...] = jnp.dot(h, w2r[...], preferred_element_type=jnp.float32) + b2r_[...]

    return pl.pallas_call(
        body,
        grid=(_B // blk,),
        in_specs=[
            pl.BlockSpec((blk, _E), lambda i: (i, 0)),
            pl.BlockSpec((blk, _E), lambda i: (i, 0)),
            pl.BlockSpec((blk, _E), lambda i: (i, 0)),
            pl.BlockSpec((_E, _H), lambda i: (0, 0)),
            pl.BlockSpec((_E, _H), lambda i: (0, 0)),
            pl.BlockSpec((_E, _H), lambda i: (0, 0)),
            pl.BlockSpec((1, _H), lambda i: (0, 0)),
            pl.BlockSpec((_H, _C), lambda i: (0, 0)),
            pl.BlockSpec((1, _C), lambda i: (0, 0)),
        ],
        out_specs=pl.BlockSpec((blk, _C), lambda i: (i, 0)),
        out_shape=jax.ShapeDtypeStruct((_B, _C), jnp.float32),
    )(sw, sb, st, w1w, w1b, w1t, b1r, w2t, b2r)


def kernel(inputs, bigram, trigram, emb_word, emb_bi, emb_tri, W1, b1, W2, b2):
    iw = inputs.astype(jnp.int32).reshape(4 * _B, _QTR)
    ib = bigram.astype(jnp.int32).reshape(4 * _B, _QTR)
    it = trigram.astype(jnp.int32).reshape(4 * _B, _QTR)
    sw, sb, st = _sc_pool(iw, ib, it, emb_word, emb_bi, emb_tri)
    return _mlp(sw, sb, st, W1, b1, W2, b2)
